# baseline (speedup 1.0000x reference)
import functools

import jax
import jax.numpy as jnp
from jax import lax
from jax.experimental import pallas as pl
from jax.experimental.pallas import tpu as pltpu
from jax.experimental.pallas import tpu_sc as plsc

N_NODES = 100000
D = 256
N_GRAPHS = 1024

N_SC = 56320
TCB = 4000
TC0 = 56000
TCG = (N_NODES - TC0) // TCB

NC = 2
NS = 16
NW = NC * NS
BLK = 80
HR = 2 * BLK
NBLK = 22
RPW = BLK * NBLK
NBLK_TOT = N_SC // BLK
NBLK_PAD = NW * NBLK
SEG_PAD = 2048
CW = 16
L = 16


NBUF = 5


NBLK_LAST = NBLK_TOT - (NW - 1) * NBLK


def _pool_body(x5_hbm, ids_hbm, idf_hbm,
               out_f, out_c,
               ids_v, idf_v, xbuf0, xbuf1, xbuf2, xbuf3, xbuf4, ones_v,
               acc_f, acc_c,
               lsem0, lsem1, lsem2, lsem3, lsem4,
               ssem0, ssem1, ssem2, ssem3, ssem4):
    c = lax.axis_index("c")
    s = lax.axis_index("s")
    wid = s * NC + c
    nblk = jnp.minimum((N_SC - wid * RPW) // BLK, NBLK)

    fseg0 = s * (SEG_PAD // NS)
    cseg0 = s * (N_GRAPHS // NS)
    bufs = ((xbuf0, lsem0, ssem0), (xbuf1, lsem1, ssem1),
            (xbuf2, lsem2, ssem2), (xbuf3, lsem3, ssem3),
            (xbuf4, lsem4, ssem4))

    for b in range(NBUF - 1):
        @pl.when(b < nblk)
        def _(b=b):
            pltpu.async_copy(x5_hbm.at[pl.ds((wid * NBLK + b) * HR, HR)],
                             bufs[b][0], bufs[b][1])
    zrow = jnp.zeros((16,), jnp.float32)

    @pl.loop(0, SEG_PAD // NS)
    def _z(i):
        for u in range(8):
            xbuf4[i, pl.ds(16 * u, 16)] = zrow

    pltpu.sync_copy(xbuf4.at[pl.ds(0, SEG_PAD // NS)],
                    acc_f.at[pl.ds(fseg0, SEG_PAD // NS)])
    pltpu.sync_copy(xbuf4.at[pl.ds(0, N_GRAPHS // NS), pl.ds(0, CW)],
                    acc_c.at[pl.ds(cseg0, N_GRAPHS // NS)])

    @pl.loop(0, BLK)
    def _ones(i):
        ones_v[i, :] = jnp.ones((CW,), jnp.float32)

    @pl.when(nblk == NBLK)
    def _():
        pltpu.sync_copy(ids_hbm.at[pl.ds(wid * NBLK, NBLK)], ids_v)
        pltpu.sync_copy(idf_hbm.at[pl.ds(wid * 2 * NBLK, 2 * NBLK)], idf_v)

    @pl.when(nblk < NBLK)
    def _():
        pltpu.sync_copy(ids_hbm.at[pl.ds(wid * NBLK, NBLK_LAST)],
                        ids_v.at[pl.ds(0, NBLK_LAST)])
        pltpu.sync_copy(idf_hbm.at[pl.ds(wid * 2 * NBLK, 2 * NBLK_LAST)],
                        idf_v.at[pl.ds(0, 2 * NBLK_LAST)])

    plsc.subcore_barrier()

    def _wait_scats(b):
        xb, _, ssem = bufs[b]
        pltpu.make_async_copy(x5_hbm.at[pl.ds(0, HR)], xb, ssem).wait()
        pltpu.make_async_copy(x5_hbm.at[pl.ds(0, BLK * CW // 128)],
                              xb.at[pl.ds(0, BLK * CW // 128)],
                              ssem).wait()

    @pl.loop(0, NBLK)
    def _blk(j):
        @pl.when(j < nblk)
        def _():
            for b in range(NBUF):
                @pl.when(j % NBUF == b)
                def _(b=b):
                    xb, lsem, ssem = bufs[b]
                    bprev = (b - 1) % NBUF
                    pltpu.make_async_copy(x5_hbm.at[pl.ds(0, HR)],
                                          xb, lsem).wait()
                    pltpu.async_copy(xb.at[pl.ds(0, BLK)],
                                     acc_f.at[idf_v.at[2 * j]], ssem,
                                     add=True)
                    pltpu.async_copy(xb.at[pl.ds(BLK, BLK)],
                                     acc_f.at[idf_v.at[2 * j + 1]], ssem,
                                     add=True)
                    pltpu.async_copy(ones_v, acc_c.at[ids_v.at[j]], ssem,
                                     add=True)
                    @pl.when(j >= 1)
                    def _(b=b):
                        _wait_scats(bprev)

                    @pl.when(j + NBUF - 1 < nblk)
                    def _(b=b):
                        ob, olsem, _ = bufs[bprev]
                        pltpu.async_copy(
                            x5_hbm.at[
                                pl.ds((wid * NBLK + j + NBUF - 1) * HR, HR)],
                            ob, olsem)

    for b in range(NBUF):
        @pl.when((nblk - 1) % NBUF == b)
        def _(b=b):
            _wait_scats(b)

    plsc.subcore_barrier()
    pltpu.sync_copy(acc_f.at[pl.ds(fseg0, SEG_PAD // NS)],
                    out_f.at[c, pl.ds(fseg0, SEG_PAD // NS)])
    pltpu.sync_copy(acc_c.at[pl.ds(cseg0, N_GRAPHS // NS)],
                    out_c.at[c, pl.ds(cseg0, N_GRAPHS // NS)])


_pool = functools.partial(
    pl.kernel,
    out_type=[
        jax.ShapeDtypeStruct((NC, SEG_PAD, 128), jnp.float32),
        jax.ShapeDtypeStruct((NC, N_GRAPHS, CW), jnp.float32),
    ],
    mesh=plsc.VectorSubcoreMesh(core_axis_name="c", subcore_axis_name="s",
                                num_cores=NC, num_subcores=NS),
    compiler_params=pltpu.CompilerParams(use_tc_tiling_on_sc=False),
    scratch_types=[
        pltpu.VMEM((NBLK, BLK), jnp.int32),
        pltpu.VMEM((2 * NBLK, BLK), jnp.int32),
        pltpu.VMEM((HR, 128), jnp.float32),
        pltpu.VMEM((HR, 128), jnp.float32),
        pltpu.VMEM((HR, 128), jnp.float32),
        pltpu.VMEM((HR, 128), jnp.float32),
        pltpu.VMEM((HR, 128), jnp.float32),
        pltpu.VMEM((BLK, CW), jnp.float32),
        pltpu.VMEM_SHARED((SEG_PAD, 128), jnp.float32),
        pltpu.VMEM_SHARED((N_GRAPHS, CW), jnp.float32),
        pltpu.SemaphoreType.DMA,
        pltpu.SemaphoreType.DMA,
        pltpu.SemaphoreType.DMA,
        pltpu.SemaphoreType.DMA,
        pltpu.SemaphoreType.DMA,
        pltpu.SemaphoreType.DMA,
        pltpu.SemaphoreType.DMA,
        pltpu.SemaphoreType.DMA,
        pltpu.SemaphoreType.DMA,
        pltpu.SemaphoreType.DMA,
    ],
)(_pool_body)


def _tcpool_body(idr_ref, xb_ref, out_ref, cnt_ref):
    j = pl.program_id(0)
    ids_blk = idr_ref[0, 0, :]
    iota2 = lax.broadcasted_iota(jnp.int32, (N_GRAPHS, TCB), 0)
    gcol = (lax.broadcasted_iota(jnp.int32, (N_GRAPHS, TCB), 1)
            + (TC0 + j * TCB))
    oh = (ids_blk[None, :] == iota2) & (gcol >= N_SC)
    part = jnp.dot(oh.astype(jnp.bfloat16), xb_ref[...].astype(jnp.bfloat16),
                   preferred_element_type=jnp.float32)
    cnt = jnp.sum(oh.astype(jnp.float32), axis=1, keepdims=True)

    @pl.when(j == 0)
    def _():
        out_ref[...] = part
        cnt_ref[...] = cnt

    @pl.when(j > 0)
    def _():
        out_ref[...] += part
        cnt_ref[...] += cnt


_tcpool = pl.pallas_call(
    _tcpool_body,
    grid=(TCG,),
    in_specs=[
        pl.BlockSpec((1, 1, TCB), lambda j: (j, 0, 0)),
        pl.BlockSpec((TCB, D), lambda j: (TC0 // TCB + j, 0)),
    ],
    out_specs=[
        pl.BlockSpec((N_GRAPHS, D), lambda j: (0, 0)),
        pl.BlockSpec((N_GRAPHS, 1), lambda j: (0, 0)),
    ],
    out_shape=[
        jax.ShapeDtypeStruct((N_GRAPHS, D), jnp.float32),
        jax.ShapeDtypeStruct((N_GRAPHS, 1), jnp.float32),
    ],
)


def _head_body(pf_ref, pc_ref, tcf_ref, tcc_ref, y_ref, w1_ref, b1_ref,
               w2_ref, b2_ref, out_ref):
    sums = pf_ref[0, :, :] + pf_ref[1, :, :] + tcf_ref[...]
    counts = pc_ref[0, :, 0:1] + pc_ref[1, :, 0:1] + tcc_ref[...]
    h_g = sums / jnp.maximum(counts, 1.0)
    h = jnp.dot(h_g, w1_ref[...], preferred_element_type=jnp.float32)
    h = jnp.maximum(h + b1_ref[...], 0.0)
    logit = jnp.dot(h, w2_ref[...], preferred_element_type=jnp.float32)
    logit = logit + b2_ref[...]
    y = y_ref[...]
    per = (jnp.maximum(logit, 0.0) - logit * y
           + jnp.log1p(jnp.exp(-jnp.abs(logit))))
    out_ref[...] = (jnp.sum(per) / float(N_GRAPHS)).reshape(1, 1)


_head = pl.pallas_call(
    _head_body,
    out_shape=jax.ShapeDtypeStruct((1, 1), jnp.float32),
)


def kernel(x, batch, y, W1, b1, W2, b2):
    x5 = (x.reshape(N_NODES // 8, 8, 2, 128)
          .transpose(0, 2, 1, 3)
          .reshape(2 * N_NODES, 128))
    ids = batch.astype(jnp.int32)
    ids_sc = ids[:N_SC]
    bids = ids_sc.reshape(NBLK_TOT, BLK)
    a2 = (ids_sc * 2).reshape(N_SC // 8, 8)
    idf = jnp.concatenate([a2, a2 + 1], axis=1).reshape(2 * NBLK_TOT, BLK)
    bt3 = ids[TC0:].reshape(TCG, 1, TCB)
    pf, pc = _pool(x5, bids, idf)
    tcf, tcc = _tcpool(bt3, x)
    pf = pf.reshape(NC, N_GRAPHS, D)
    loss = _head(pf, pc, tcf, tcc, y, W1, b1.reshape(1, D),
                 W2, b2.reshape(1, 1))
    return loss[0, 0]


# speedup vs baseline: 1.2718x; 1.2718x over previous
import functools

import jax
import jax.numpy as jnp
from jax import lax
from jax.experimental import pallas as pl
from jax.experimental.pallas import tpu as pltpu
from jax.experimental.pallas import tpu_sc as plsc

N_NODES = 100000
D = 256
N_GRAPHS = 1024

N_SC = 64000
TCB = 4000
TCG = (N_NODES - N_SC) // TCB

NC = 2
NS = 16
NW = NC * NS
BLK = 80
HR = 2 * BLK
NBLK = 25
RPW = BLK * NBLK
NBLK_TOT = N_SC // BLK
NBLK_PAD = NW * NBLK
SEG_PAD = 2048
CW = 16
L = 16


NBUF = 5


NBLK_LAST = NBLK_TOT - (NW - 1) * NBLK


def _pool_body(x5_hbm, ids_hbm, idf_hbm,
               out_f, out_c,
               ids_v, idf_v, xbuf0, xbuf1, xbuf2, xbuf3, xbuf4, ones_v,
               acc_f, acc_c,
               lsem0, lsem1, lsem2, lsem3, lsem4,
               ssem0, ssem1, ssem2, ssem3, ssem4):
    c = lax.axis_index("c")
    s = lax.axis_index("s")
    wid = s * NC + c
    nblk = jnp.minimum((N_SC - wid * RPW) // BLK, NBLK)

    fseg0 = s * (SEG_PAD // NS)
    cseg0 = s * (N_GRAPHS // NS)
    bufs = ((xbuf0, lsem0, ssem0), (xbuf1, lsem1, ssem1),
            (xbuf2, lsem2, ssem2), (xbuf3, lsem3, ssem3),
            (xbuf4, lsem4, ssem4))

    for b in range(NBUF - 1):
        @pl.when(b < nblk)
        def _(b=b):
            pltpu.async_copy(x5_hbm.at[pl.ds((wid * NBLK + b) * HR, HR)],
                             bufs[b][0], bufs[b][1])
    zrow = jnp.zeros((16,), jnp.float32)

    @pl.loop(0, SEG_PAD // NS)
    def _z(i):
        for u in range(8):
            xbuf4[i, pl.ds(16 * u, 16)] = zrow

    pltpu.sync_copy(xbuf4.at[pl.ds(0, SEG_PAD // NS)],
                    acc_f.at[pl.ds(fseg0, SEG_PAD // NS)])
    pltpu.sync_copy(xbuf4.at[pl.ds(0, N_GRAPHS // NS), pl.ds(0, CW)],
                    acc_c.at[pl.ds(cseg0, N_GRAPHS // NS)])

    @pl.loop(0, BLK)
    def _ones(i):
        ones_v[i, :] = jnp.ones((CW,), jnp.float32)

    @pl.when(nblk == NBLK)
    def _():
        pltpu.sync_copy(ids_hbm.at[pl.ds(wid * NBLK, NBLK)], ids_v)
        pltpu.sync_copy(idf_hbm.at[pl.ds(wid * 2 * NBLK, 2 * NBLK)], idf_v)

    @pl.when(nblk < NBLK)
    def _():
        pltpu.sync_copy(ids_hbm.at[pl.ds(wid * NBLK, NBLK_LAST)],
                        ids_v.at[pl.ds(0, NBLK_LAST)])
        pltpu.sync_copy(idf_hbm.at[pl.ds(wid * 2 * NBLK, 2 * NBLK_LAST)],
                        idf_v.at[pl.ds(0, 2 * NBLK_LAST)])

    plsc.subcore_barrier()

    def _wait_scats(b):
        xb, _, ssem = bufs[b]
        pltpu.make_async_copy(x5_hbm.at[pl.ds(0, HR)], xb, ssem).wait()
        pltpu.make_async_copy(x5_hbm.at[pl.ds(0, BLK * CW // 128)],
                              xb.at[pl.ds(0, BLK * CW // 128)],
                              ssem).wait()

    @pl.loop(0, NBLK)
    def _blk(j):
        @pl.when(j < nblk)
        def _():
            for b in range(NBUF):
                @pl.when(j % NBUF == b)
                def _(b=b):
                    xb, lsem, ssem = bufs[b]
                    bprev = (b - 1) % NBUF
                    pltpu.make_async_copy(x5_hbm.at[pl.ds(0, HR)],
                                          xb, lsem).wait()
                    pltpu.async_copy(xb.at[pl.ds(0, BLK)],
                                     acc_f.at[idf_v.at[2 * j]], ssem,
                                     add=True)
                    pltpu.async_copy(xb.at[pl.ds(BLK, BLK)],
                                     acc_f.at[idf_v.at[2 * j + 1]], ssem,
                                     add=True)
                    pltpu.async_copy(ones_v, acc_c.at[ids_v.at[j]], ssem,
                                     add=True)
                    @pl.when(j >= 1)
                    def _(b=b):
                        _wait_scats(bprev)

                    @pl.when(j + NBUF - 1 < nblk)
                    def _(b=b):
                        ob, olsem, _ = bufs[bprev]
                        pltpu.async_copy(
                            x5_hbm.at[
                                pl.ds((wid * NBLK + j + NBUF - 1) * HR, HR)],
                            ob, olsem)

    for b in range(NBUF):
        @pl.when((nblk - 1) % NBUF == b)
        def _(b=b):
            _wait_scats(b)

    plsc.subcore_barrier()
    pltpu.sync_copy(acc_f.at[pl.ds(fseg0, SEG_PAD // NS)],
                    out_f.at[c, pl.ds(fseg0, SEG_PAD // NS)])
    pltpu.sync_copy(acc_c.at[pl.ds(cseg0, N_GRAPHS // NS)],
                    out_c.at[c, pl.ds(cseg0, N_GRAPHS // NS)])


_pool = functools.partial(
    pl.kernel,
    out_type=[
        jax.ShapeDtypeStruct((NC, SEG_PAD, 128), jnp.float32),
        jax.ShapeDtypeStruct((NC, N_GRAPHS, CW), jnp.float32),
    ],
    mesh=plsc.VectorSubcoreMesh(core_axis_name="c", subcore_axis_name="s",
                                num_cores=NC, num_subcores=NS),
    compiler_params=pltpu.CompilerParams(use_tc_tiling_on_sc=False),
    scratch_types=[
        pltpu.VMEM((NBLK, BLK), jnp.int32),
        pltpu.VMEM((2 * NBLK, BLK), jnp.int32),
        pltpu.VMEM((HR, 128), jnp.float32),
        pltpu.VMEM((HR, 128), jnp.float32),
        pltpu.VMEM((HR, 128), jnp.float32),
        pltpu.VMEM((HR, 128), jnp.float32),
        pltpu.VMEM((HR, 128), jnp.float32),
        pltpu.VMEM((BLK, CW), jnp.float32),
        pltpu.VMEM_SHARED((SEG_PAD, 128), jnp.float32),
        pltpu.VMEM_SHARED((N_GRAPHS, CW), jnp.float32),
        pltpu.SemaphoreType.DMA,
        pltpu.SemaphoreType.DMA,
        pltpu.SemaphoreType.DMA,
        pltpu.SemaphoreType.DMA,
        pltpu.SemaphoreType.DMA,
        pltpu.SemaphoreType.DMA,
        pltpu.SemaphoreType.DMA,
        pltpu.SemaphoreType.DMA,
        pltpu.SemaphoreType.DMA,
        pltpu.SemaphoreType.DMA,
    ],
)(_pool_body)


def _tcpool_body(idr_ref, xb_ref, out_ref, cnt_ref):
    j = pl.program_id(0)
    ids_blk = idr_ref[0, 0, :]
    iota2 = lax.broadcasted_iota(jnp.int32, (N_GRAPHS, TCB), 0)
    oh = ids_blk[None, :] == iota2
    part = jnp.dot(oh.astype(jnp.bfloat16), xb_ref[...].astype(jnp.bfloat16),
                   preferred_element_type=jnp.float32)
    cnt = jnp.sum(oh.astype(jnp.float32), axis=1, keepdims=True)

    @pl.when(j == 0)
    def _():
        out_ref[...] = part
        cnt_ref[...] = cnt

    @pl.when(j > 0)
    def _():
        out_ref[...] += part
        cnt_ref[...] += cnt


_tcpool = pl.pallas_call(
    _tcpool_body,
    grid=(TCG,),
    in_specs=[
        pl.BlockSpec((1, 1, TCB), lambda j: (j, 0, 0)),
        pl.BlockSpec((TCB, D), lambda j: (N_SC // TCB + j, 0)),
    ],
    out_specs=[
        pl.BlockSpec((N_GRAPHS, D), lambda j: (0, 0)),
        pl.BlockSpec((N_GRAPHS, 1), lambda j: (0, 0)),
    ],
    out_shape=[
        jax.ShapeDtypeStruct((N_GRAPHS, D), jnp.float32),
        jax.ShapeDtypeStruct((N_GRAPHS, 1), jnp.float32),
    ],
)


def _head_body(pf_ref, pc_ref, tcf_ref, tcc_ref, y_ref, w1_ref, b1_ref,
               w2_ref, b2_ref, out_ref):
    sums = pf_ref[0, :, :] + pf_ref[1, :, :] + tcf_ref[...]
    counts = pc_ref[0, :, 0:1] + pc_ref[1, :, 0:1] + tcc_ref[...]
    h_g = sums / jnp.maximum(counts, 1.0)
    h = jnp.dot(h_g, w1_ref[...], preferred_element_type=jnp.float32)
    h = jnp.maximum(h + b1_ref[...], 0.0)
    logit = jnp.dot(h, w2_ref[...], preferred_element_type=jnp.float32)
    logit = logit + b2_ref[...]
    y = y_ref[...]
    per = (jnp.maximum(logit, 0.0) - logit * y
           + jnp.log1p(jnp.exp(-jnp.abs(logit))))
    out_ref[...] = (jnp.sum(per) / float(N_GRAPHS)).reshape(1, 1)


_head = pl.pallas_call(
    _head_body,
    out_shape=jax.ShapeDtypeStruct((1, 1), jnp.float32),
)


def kernel(x, batch, y, W1, b1, W2, b2):
    x5 = (x.reshape(N_NODES // 8, 8, 2, 128)
          .transpose(0, 2, 1, 3)
          .reshape(2 * N_NODES, 128))
    ids = batch.astype(jnp.int32)
    ids_sc = ids[:N_SC]
    bids = ids_sc.reshape(NBLK_TOT, BLK)
    a2 = (ids_sc * 2).reshape(N_SC // 8, 8)
    idf = jnp.concatenate([a2, a2 + 1], axis=1).reshape(2 * NBLK_TOT, BLK)
    bt3 = ids[N_SC:].reshape(TCG, 1, TCB)
    pf, pc = _pool(x5, bids, idf)
    tcf, tcc = _tcpool(bt3, x)
    pf = pf.reshape(NC, N_GRAPHS, D)
    loss = _head(pf, pc, tcf, tcc, y, W1, b1.reshape(1, D),
                 W2, b2.reshape(1, 1))
    return loss[0, 0]
